# baseline (speedup 1.0000x reference)
import functools

import jax
import jax.numpy as jnp
from jax import lax
from jax.experimental import pallas as pl
from jax.experimental.pallas import tpu as pltpu
from jax.experimental.pallas import tpu_sc as plsc

NC = 2
NS = 16
NW = NC * NS
L = 16


def _mf_body(B, D, uidx_hbm, iidx_hbm, utab_hbm, itab_hbm, ub_hbm, ib_hbm,
             glob_hbm, out_hbm, uidx_v, iidx_v, urows_v, irows_v, ub_v, ib_v,
             out_v, glob_v, sem):
    bpw = B // NW
    wid = lax.axis_index("s") * NC + lax.axis_index("c")
    base = wid * bpw

    pltpu.sync_copy(uidx_hbm.at[pl.ds(base, bpw)], uidx_v)
    pltpu.sync_copy(iidx_hbm.at[pl.ds(base, bpw)], iidx_v)
    pltpu.sync_copy(glob_hbm, glob_v)

    c1 = pltpu.async_copy(utab_hbm.at[uidx_v], urows_v, sem)
    c2 = pltpu.async_copy(itab_hbm.at[iidx_v], irows_v, sem)
    c3 = pltpu.async_copy(ub_hbm.at[uidx_v], ub_v, sem)
    c4 = pltpu.async_copy(ib_hbm.at[iidx_v], ib_v, sem)
    c1.wait()
    c2.wait()
    c3.wait()
    c4.wait()

    gvec = glob_v[...]
    iota = lax.iota(jnp.int32, L)
    zero16 = jnp.zeros((L,), jnp.int32)

    def group(g, carry):
        e0 = g * L
        rid = iota + e0
        acc = (plsc.load_gather(ub_v, [rid, zero16])
               + plsc.load_gather(ib_v, [rid, zero16]) + gvec)
        for d in range(D):
            cid = jnp.full((L,), d, jnp.int32)
            u = plsc.load_gather(urows_v, [rid, cid])
            it = plsc.load_gather(irows_v, [rid, cid])
            acc = acc + u * it
        out_v[pl.ds(e0, L)] = acc
        return carry

    lax.fori_loop(0, bpw // L, group, 0)

    pltpu.sync_copy(out_v, out_hbm.at[pl.ds(base, bpw)])


def kernel(user_indices, item_indices, user_table, item_table, user_bias,
           item_bias, global_rating):
    B = user_indices.shape[0]
    D = user_table.shape[1]
    bpw = B // NW

    uidx = user_indices.astype(jnp.int32)
    iidx = item_indices.astype(jnp.int32)
    glob = jnp.broadcast_to(global_rating.astype(jnp.float32), (L,))

    mesh = plsc.VectorSubcoreMesh(core_axis_name="c", subcore_axis_name="s")
    f = pl.kernel(
        functools.partial(_mf_body, B, D),
        out_type=jax.ShapeDtypeStruct((B,), jnp.float32),
        mesh=mesh,
        compiler_params=pltpu.CompilerParams(
            needs_layout_passes=False, use_tc_tiling_on_sc=False),
        scratch_types=[
            pltpu.VMEM((bpw,), jnp.int32),
            pltpu.VMEM((bpw,), jnp.int32),
            pltpu.VMEM((bpw, D), jnp.float32),
            pltpu.VMEM((bpw, D), jnp.float32),
            pltpu.VMEM((bpw, 1), jnp.float32),
            pltpu.VMEM((bpw, 1), jnp.float32),
            pltpu.VMEM((bpw,), jnp.float32),
            pltpu.VMEM((L,), jnp.float32),
            pltpu.SemaphoreType.DMA,
        ],
    )
    return f(uidx, iidx, user_table, item_table, user_bias, item_bias, glob)


# speedup vs baseline: 2.8827x; 2.8827x over previous
import functools

import jax
import jax.numpy as jnp
from jax import lax
from jax.experimental import pallas as pl
from jax.experimental.pallas import tpu as pltpu
from jax.experimental.pallas import tpu_sc as plsc

NC = 2
NS = 16
NW = NC * NS
L = 16


def _mf_body(B, D, uidx_hbm, iidx_hbm, utab_hbm, itab_hbm, ub_hbm, ib_hbm,
             glob_hbm, out_hbm, uidx_v, iidx_v, urows_v, irows_v, ub_v, ib_v,
             out_v, glob_v, sem):
    bpw = B // NW
    wid = lax.axis_index("s") * NC + lax.axis_index("c")
    base = wid * bpw

    pltpu.sync_copy(uidx_hbm.at[pl.ds(base, bpw)], uidx_v)
    pltpu.sync_copy(iidx_hbm.at[pl.ds(base, bpw)], iidx_v)
    pltpu.sync_copy(glob_hbm, glob_v)

    c1 = pltpu.async_copy(utab_hbm.at[uidx_v], urows_v, sem)
    c2 = pltpu.async_copy(itab_hbm.at[iidx_v], irows_v, sem)
    c3 = pltpu.async_copy(ub_hbm.at[uidx_v], ub_v, sem)
    c4 = pltpu.async_copy(ib_hbm.at[iidx_v], ib_v, sem)
    c1.wait()
    c2.wait()
    c3.wait()
    c4.wait()

    gvec = glob_v[...]
    iota = lax.iota(jnp.int32, L)

    def group(g, carry):
        e0 = g * L
        acc = ub_v[pl.ds(e0, L)] + ib_v[pl.ds(e0, L)] + gvec
        rid = iota + e0
        for d in range(D):
            cid = jnp.full((L,), d, jnp.int32)
            u = plsc.load_gather(urows_v, [rid, cid])
            it = plsc.load_gather(irows_v, [rid, cid])
            acc = acc + u * it
        out_v[pl.ds(e0, L)] = acc
        return carry

    lax.fori_loop(0, bpw // L, group, 0)

    pltpu.sync_copy(out_v, out_hbm.at[pl.ds(base, bpw)])


def kernel(user_indices, item_indices, user_table, item_table, user_bias,
           item_bias, global_rating):
    B = user_indices.shape[0]
    D = user_table.shape[1]
    bpw = B // NW

    uidx = user_indices.astype(jnp.int32)
    iidx = item_indices.astype(jnp.int32)
    ub = user_bias.reshape(-1)
    ib = item_bias.reshape(-1)
    glob = jnp.broadcast_to(global_rating.astype(jnp.float32), (L,))

    mesh = plsc.VectorSubcoreMesh(core_axis_name="c", subcore_axis_name="s")
    f = pl.kernel(
        functools.partial(_mf_body, B, D),
        out_type=jax.ShapeDtypeStruct((B,), jnp.float32),
        mesh=mesh,
        compiler_params=pltpu.CompilerParams(
            needs_layout_passes=False, use_tc_tiling_on_sc=False),
        scratch_types=[
            pltpu.VMEM((bpw,), jnp.int32),
            pltpu.VMEM((bpw,), jnp.int32),
            pltpu.VMEM((bpw, D), jnp.float32),
            pltpu.VMEM((bpw, D), jnp.float32),
            pltpu.VMEM((bpw,), jnp.float32),
            pltpu.VMEM((bpw,), jnp.float32),
            pltpu.VMEM((bpw,), jnp.float32),
            pltpu.VMEM((L,), jnp.float32),
            pltpu.SemaphoreType.DMA,
        ],
    )
    return f(uidx, iidx, user_table, item_table, ub, ib, glob)
